# Initial kernel scaffold; baseline (speedup 1.0000x reference)
import jax
import jax.numpy as jnp
from jax.experimental import pallas as pl


def kernel(pos, edge_index, W_in, Ws, w_out, We1, be1, We2, Wf_in, bf_in, Wf_h, bf_h, Wf_out, bf_out):
    raise NotImplementedError("write your pallas kernel here")


# speedup vs baseline: 1.0323x; 1.0323x over previous
import functools

import jax
import jax.numpy as jnp
import numpy as np
from jax.experimental import pallas as pl
from jax.experimental.pallas import tpu as pltpu

N = 10000
E = 160000
RATIO = 0.5
K = 8
NS = int((1 - RATIO) * N)
PS_LAYERS = 3
PS_OUT = 128
EP_HID = 128
FC_HID = 256
FC_LAYERS = 2
_JJ, _LL = np.triu_indices(K, 1)
P = _JJ.shape[0]
F = NS * P

_FBLK = 1024
FP = ((F + _FBLK - 1) // _FBLK) * _FBLK


def _face_mlp_body(x_ref, w_in_ref, b_in_ref, wh0_ref, bh0_ref, wh1_ref,
                   bh1_ref, w_out_ref, b_out_ref, o_ref):
    x = x_ref[...]
    fmask = x[10:11, :]
    dn = (((0,), (0,)), ((), ()))
    h = jax.lax.dot_general(w_in_ref[...], x, dn,
                            preferred_element_type=jnp.float32)
    h = jnp.maximum(h + b_in_ref[...].reshape(FC_HID, 1), 0.0)
    h = jax.lax.dot_general(wh0_ref[...], h, dn,
                            preferred_element_type=jnp.float32)
    h = jnp.maximum(h + bh0_ref[...].reshape(FC_HID, 1), 0.0)
    h = jax.lax.dot_general(wh1_ref[...], h, dn,
                            preferred_element_type=jnp.float32)
    h = jnp.maximum(h + bh1_ref[...].reshape(FC_HID, 1), 0.0)
    logit = jax.lax.dot_general(w_out_ref[...], h, dn,
                                preferred_element_type=jnp.float32)
    o_ref[...] = jax.nn.sigmoid(logit + b_out_ref[0, 0]) * fmask


def _face_mlp(x0t, Wf_in, bf_in, Wf_h, bf_h, Wf_out, bf_out):
    w_in_p = jnp.zeros((16, FC_HID), jnp.float32).at[:10].set(Wf_in)
    grid = (FP // _FBLK,)
    out = pl.pallas_call(
        _face_mlp_body,
        grid=grid,
        in_specs=[
            pl.BlockSpec((16, _FBLK), lambda i: (0, i)),
            pl.BlockSpec((16, FC_HID), lambda i: (0, 0)),
            pl.BlockSpec((FC_HID,), lambda i: (0,)),
            pl.BlockSpec((FC_HID, FC_HID), lambda i: (0, 0)),
            pl.BlockSpec((FC_HID,), lambda i: (0,)),
            pl.BlockSpec((FC_HID, FC_HID), lambda i: (0, 0)),
            pl.BlockSpec((FC_HID,), lambda i: (0,)),
            pl.BlockSpec((FC_HID, 1), lambda i: (0, 0)),
            pl.BlockSpec((1, 1), lambda i: (0, 0)),
        ],
        out_specs=pl.BlockSpec((1, _FBLK), lambda i: (0, i)),
        out_shape=jax.ShapeDtypeStruct((1, FP), jnp.float32),
    )(x0t, w_in_p, bf_in, Wf_h[0], bf_h[0], Wf_h[1], bf_h[1], Wf_out,
      bf_out.reshape(1, 1))
    return out[0]


def kernel(pos, edge_index, W_in, Ws, w_out, We1, be1, We2, Wf_in, bf_in,
           Wf_h, bf_h, Wf_out, bf_out):
    src = edge_index[0]
    dst = edge_index[1]
    h = jnp.tanh(pos @ W_in)
    deg = jnp.clip(jax.ops.segment_sum(jnp.ones((E,), jnp.float32), dst,
                                       num_segments=N), 1.0)
    for i in range(PS_LAYERS):
        agg = jax.ops.segment_sum(h[src], dst, num_segments=N) / deg[:, None]
        h = jnp.tanh((h + agg) @ Ws[i])
    sampled_probs = jax.nn.sigmoid(h @ w_out)[:, 0]
    _, sampled_indices = jax.lax.top_k(sampled_probs, NS)
    sampled_pos = pos[sampled_indices]
    new_id = jnp.full((N,), -1, jnp.int32).at[sampled_indices].set(
        jnp.arange(NS, dtype=jnp.int32))
    s2 = new_id[src]
    d2 = new_id[dst]
    emask = ((s2 >= 0) & (d2 >= 0)).astype(jnp.float32)
    s2c = jnp.where(s2 >= 0, s2, 0)
    d2c = jnp.where(d2 >= 0, d2, 0)
    ef = jnp.concatenate([sampled_pos[s2c], sampled_pos[d2c]], axis=1)
    eprob = jax.nn.sigmoid(jax.nn.relu(ef @ We1 + be1) @ We2)[:, 0] * emask
    adj = jnp.zeros((NS, NS), jnp.float32).at[s2c, d2c].max(eprob)
    vals, knn = jax.lax.top_k(adj, K)
    jj = jnp.asarray(_JJ, jnp.int32)
    ll = jnp.asarray(_LL, jnp.int32)
    n1 = knn[:, jj]
    n2 = knn[:, ll]
    a12 = adj[n1, n2]
    tmask = (a12 > 0).astype(jnp.float32)
    tri_prob = (vals[:, jj] * vals[:, ll] * a12 / 3.0) * tmask
    ii = jnp.broadcast_to(jnp.arange(NS, dtype=jnp.int32)[:, None], n1.shape)
    faces = jnp.stack([ii, n1, n2], axis=-1).reshape(-1, 3)
    cprobs = tri_prob.reshape(-1)
    fmask = tmask.reshape(-1)
    fp = sampled_pos[faces].reshape(-1, 9)
    x0t = jnp.zeros((16, FP), jnp.float32)
    x0t = x0t.at[:9, :F].set(fp.T)
    x0t = x0t.at[9, :F].set(cprobs)
    x0t = x0t.at[10, :F].set(fmask)
    face_probs = _face_mlp(x0t, Wf_in, bf_in, Wf_h, bf_h, Wf_out,
                           bf_out)[:F]
    threshold = jnp.quantile(face_probs, 1.0 - RATIO)
    simplified_mask = (face_probs > threshold).astype(jnp.int32)
    return (sampled_probs, face_probs, simplified_mask)


# speedup vs baseline: 1.1558x; 1.1196x over previous
import functools

import jax
import jax.numpy as jnp
import numpy as np
from jax.experimental import pallas as pl
from jax.experimental.pallas import tpu as pltpu

N = 10000
E = 160000
RATIO = 0.5
K = 8
NS = int((1 - RATIO) * N)
PS_LAYERS = 3
PS_OUT = 128
EP_HID = 128
FC_HID = 256
FC_LAYERS = 2
_JJ, _LL = np.triu_indices(K, 1)
P = _JJ.shape[0]
F = NS * P

_FBLK = 1024
FP = ((F + _FBLK - 1) // _FBLK) * _FBLK


_RB = 200


def _topk_body(adj_ref, knn_ref, vals_ref):
    v = adj_ref[...]
    iota = jax.lax.broadcasted_iota(jnp.int32, (_RB, NS), 1)
    knn_cols, vals_cols = [], []
    for _ in range(K):
        m = jnp.max(v, axis=1, keepdims=True)
        idx = jnp.min(jnp.where(v == m, iota, jnp.int32(NS)), axis=1,
                      keepdims=True)
        knn_cols.append(idx)
        vals_cols.append(m)
        v = jnp.where(iota == idx, -jnp.inf, v)
    knn_ref[...] = jnp.concatenate(knn_cols, 1)
    vals_ref[...] = jnp.concatenate(vals_cols, 1)


def _row_topk(adj):
    return pl.pallas_call(
        _topk_body,
        grid=(NS // _RB,),
        in_specs=[pl.BlockSpec((_RB, NS), lambda i: (i, 0))],
        out_specs=[pl.BlockSpec((_RB, K), lambda i: (i, 0)),
                   pl.BlockSpec((_RB, K), lambda i: (i, 0))],
        out_shape=[jax.ShapeDtypeStruct((NS, K), jnp.int32),
                   jax.ShapeDtypeStruct((NS, K), jnp.float32)],
    )(adj)


_QK = [F // 2 - 1, F // 2]


def _quantile_body(fp_ref, mask_ref):
    v = fp_ref[...]
    u = jax.lax.bitcast_convert_type(v, jnp.uint32)

    def kth(k):
        def step(i, p):
            b = 31 - i
            mid = p | (jnp.uint32(1) << b.astype(jnp.uint32))
            c = jnp.sum((u < mid).astype(jnp.int32))
            return jnp.where(c > k, p, mid)
        return jax.lax.fori_loop(0, 32, step, jnp.uint32(0))

    lo = jax.lax.bitcast_convert_type(kth(_QK[0]), jnp.float32)
    hi = jax.lax.bitcast_convert_type(kth(_QK[1]), jnp.float32)
    thr = 0.5 * lo + 0.5 * hi
    mask_ref[...] = (v > thr).astype(jnp.int32)


def _quantile_mask(face_probs_padded):
    fp2 = face_probs_padded.reshape(FP // 128, 128)
    out = pl.pallas_call(
        _quantile_body,
        out_shape=jax.ShapeDtypeStruct((FP // 128, 128), jnp.int32),
    )(fp2)
    return out.reshape(FP)[:F]


def _face_mlp_body(x_ref, w_in_ref, b_in_ref, wh0_ref, bh0_ref, wh1_ref,
                   bh1_ref, w_out_ref, b_out_ref, o_ref):
    x = x_ref[...]
    fmask = x[10:11, :]
    dn = (((0,), (0,)), ((), ()))
    h = jax.lax.dot_general(w_in_ref[...], x, dn,
                            preferred_element_type=jnp.float32)
    h = jnp.maximum(h + b_in_ref[...].reshape(FC_HID, 1), 0.0)
    h = jax.lax.dot_general(wh0_ref[...], h, dn,
                            preferred_element_type=jnp.float32)
    h = jnp.maximum(h + bh0_ref[...].reshape(FC_HID, 1), 0.0)
    h = jax.lax.dot_general(wh1_ref[...], h, dn,
                            preferred_element_type=jnp.float32)
    h = jnp.maximum(h + bh1_ref[...].reshape(FC_HID, 1), 0.0)
    logit = jax.lax.dot_general(w_out_ref[...], h, dn,
                                preferred_element_type=jnp.float32)
    o_ref[...] = jax.nn.sigmoid(logit + b_out_ref[0, 0]) * fmask


def _face_mlp(x0t, Wf_in, bf_in, Wf_h, bf_h, Wf_out, bf_out):
    w_in_p = jnp.zeros((16, FC_HID), jnp.float32).at[:10].set(Wf_in)
    grid = (FP // _FBLK,)
    out = pl.pallas_call(
        _face_mlp_body,
        grid=grid,
        in_specs=[
            pl.BlockSpec((16, _FBLK), lambda i: (0, i)),
            pl.BlockSpec((16, FC_HID), lambda i: (0, 0)),
            pl.BlockSpec((FC_HID,), lambda i: (0,)),
            pl.BlockSpec((FC_HID, FC_HID), lambda i: (0, 0)),
            pl.BlockSpec((FC_HID,), lambda i: (0,)),
            pl.BlockSpec((FC_HID, FC_HID), lambda i: (0, 0)),
            pl.BlockSpec((FC_HID,), lambda i: (0,)),
            pl.BlockSpec((FC_HID, 1), lambda i: (0, 0)),
            pl.BlockSpec((1, 1), lambda i: (0, 0)),
        ],
        out_specs=pl.BlockSpec((1, _FBLK), lambda i: (0, i)),
        out_shape=jax.ShapeDtypeStruct((1, FP), jnp.float32),
    )(x0t, w_in_p, bf_in, Wf_h[0], bf_h[0], Wf_h[1], bf_h[1], Wf_out,
      bf_out.reshape(1, 1))
    return out[0]


def kernel(pos, edge_index, W_in, Ws, w_out, We1, be1, We2, Wf_in, bf_in,
           Wf_h, bf_h, Wf_out, bf_out):
    src = edge_index[0]
    dst = edge_index[1]
    h = jnp.tanh(pos @ W_in)
    deg = jnp.clip(jax.ops.segment_sum(jnp.ones((E,), jnp.float32), dst,
                                       num_segments=N), 1.0)
    for i in range(PS_LAYERS):
        agg = jax.ops.segment_sum(h[src], dst, num_segments=N) / deg[:, None]
        h = jnp.tanh((h + agg) @ Ws[i])
    sampled_probs = jax.nn.sigmoid(h @ w_out)[:, 0]
    _, sampled_indices = jax.lax.top_k(sampled_probs, NS)
    sampled_pos = pos[sampled_indices]
    new_id = jnp.full((N,), -1, jnp.int32).at[sampled_indices].set(
        jnp.arange(NS, dtype=jnp.int32))
    s2 = new_id[src]
    d2 = new_id[dst]
    emask = ((s2 >= 0) & (d2 >= 0)).astype(jnp.float32)
    s2c = jnp.where(s2 >= 0, s2, 0)
    d2c = jnp.where(d2 >= 0, d2, 0)
    ef = jnp.concatenate([sampled_pos[s2c], sampled_pos[d2c]], axis=1)
    eprob = jax.nn.sigmoid(jax.nn.relu(ef @ We1 + be1) @ We2)[:, 0] * emask
    adj = jnp.zeros((NS, NS), jnp.float32).at[s2c, d2c].max(eprob)
    knn, vals = _row_topk(adj)
    jj = jnp.asarray(_JJ, jnp.int32)
    ll = jnp.asarray(_LL, jnp.int32)
    n1 = knn[:, jj]
    n2 = knn[:, ll]
    a12 = adj[n1, n2]
    tmask = (a12 > 0).astype(jnp.float32)
    tri_prob = (vals[:, jj] * vals[:, ll] * a12 / 3.0) * tmask
    ii = jnp.broadcast_to(jnp.arange(NS, dtype=jnp.int32)[:, None], n1.shape)
    faces = jnp.stack([ii, n1, n2], axis=-1).reshape(-1, 3)
    cprobs = tri_prob.reshape(-1)
    fmask = tmask.reshape(-1)
    fp = sampled_pos[faces].reshape(-1, 9)
    x0t = jnp.zeros((16, FP), jnp.float32)
    x0t = x0t.at[:9, :F].set(fp.T)
    x0t = x0t.at[9, :F].set(cprobs)
    x0t = x0t.at[10, :F].set(fmask)
    face_probs_p = _face_mlp(x0t, Wf_in, bf_in, Wf_h, bf_h, Wf_out, bf_out)
    face_probs = face_probs_p[:F]
    fp_pad = jnp.where(
        jnp.arange(FP, dtype=jnp.int32) < F, face_probs_p, jnp.inf)
    simplified_mask = _quantile_mask(fp_pad)
    return (sampled_probs, face_probs, simplified_mask)


# speedup vs baseline: 1.2666x; 1.0959x over previous
import functools

import jax
import jax.numpy as jnp
import numpy as np
from jax import lax
from jax.experimental import pallas as pl
from jax.experimental.pallas import tpu as pltpu
from jax.experimental.pallas import tpu_sc as plsc

N = 10000
E = 160000
RATIO = 0.5
K = 8
NS = int((1 - RATIO) * N)
PS_LAYERS = 3
PS_OUT = 128
EP_HID = 128
FC_HID = 256
FC_LAYERS = 2
_JJ, _LL = np.triu_indices(K, 1)
P = _JJ.shape[0]
F = NS * P

_FBLK = 1024
FP = ((F + _FBLK - 1) // _FBLK) * _FBLK


_NC, _NSUB, _NLANE = 2, 16, 16
_NW = _NC * _NSUB
_ECH = 640
_NCHUNK = E // _ECH
_NP = 10240
_RPT = _NP // _NSUB


_HALF = 5120
_HALFP = _HALF + 8
_RPT = _HALF // _NSUB
_CPT = _NCHUNK // _NSUB


def _gather_rows_body(h_hbm, src_hbm, out_hbm, idx_v, rows_v, sem):
    c = lax.axis_index("c")
    s = lax.axis_index("s")
    wid = s * _NC + c

    nch = jnp.where(wid < _NCHUNK % _NW, _NCHUNK // _NW + 1, _NCHUNK // _NW)

    def chunk_body(i, carry):
        base = (wid + i * _NW) * _ECH
        pltpu.sync_copy(src_hbm.at[pl.ds(base, _ECH)], idx_v)
        pltpu.async_copy(h_hbm.at[idx_v], rows_v, sem).wait()
        pltpu.sync_copy(rows_v, out_hbm.at[pl.ds(base, _ECH)])
        return carry

    lax.fori_loop(0, nch, chunk_body, 0)


def _sc_gather_rows(h, src):
    mesh = plsc.VectorSubcoreMesh(core_axis_name="c", subcore_axis_name="s")
    fn = pl.kernel(
        _gather_rows_body,
        out_type=jax.ShapeDtypeStruct((E, PS_OUT), jnp.float32),
        mesh=mesh,
        scratch_types=[
            pltpu.VMEM((_ECH,), jnp.int32),
            pltpu.VMEM((_ECH, PS_OUT), jnp.float32),
            pltpu.SemaphoreType.DMA,
        ],
    )
    return fn(h, src)


def _deg_body(dst_hbm, zdeg_hbm, ones_hbm, deg_hbm, dst_v, ones_v, degz_v,
              degacc_sh, sem):
    c = lax.axis_index("c")
    s = lax.axis_index("s")
    wid = s * _NC + c
    @pl.when(s == 0)
    def _():
        pltpu.sync_copy(zdeg_hbm, degz_v)
        pltpu.sync_copy(degz_v, degacc_sh)
    pltpu.sync_copy(ones_hbm, ones_v)
    plsc.subcore_barrier()

    nch = jnp.where(wid < _NCHUNK % _NW, _NCHUNK // _NW + 1, _NCHUNK // _NW)

    def chunk_body(i, carry):
        base = (wid + i * _NW) * _ECH
        pltpu.sync_copy(dst_hbm.at[pl.ds(base, _ECH)], dst_v)
        pltpu.sync_copy(ones_v, degacc_sh.at[dst_v], add=True)
        return carry

    lax.fori_loop(0, nch, chunk_body, 0)
    plsc.subcore_barrier()
    @pl.when(s == 0)
    def _():
        pltpu.sync_copy(degacc_sh, degz_v)
        pltpu.sync_copy(degz_v, deg_hbm.at[c, 0])


def _sc_deg(dst):
    mesh = plsc.VectorSubcoreMesh(core_axis_name="c", subcore_axis_name="s")
    zdeg = jnp.zeros((N + 16,), jnp.float32)
    ones = jnp.ones((_ECH,), jnp.float32)
    fn = pl.kernel(
        _deg_body,
        out_type=jax.ShapeDtypeStruct((2, 1, N + 16), jnp.float32),
        mesh=mesh,
        scratch_types=[
            pltpu.VMEM((_ECH,), jnp.int32),
            pltpu.VMEM((_ECH,), jnp.float32),
            pltpu.VMEM((N + 16,), jnp.float32),
            pltpu.VMEM_SHARED((N + 16,), jnp.float32),
            pltpu.SemaphoreType.DMA,
        ],
    )
    return fn(dst, zdeg, ones)


_GB = 1000


def _h0_body(pos_ref, w_ref, o_ref):
    o_ref[...] = jnp.tanh(
        jax.lax.dot_general(pos_ref[...], w_ref[...], (((1,), (0,)), ((), ())),
                            preferred_element_type=jnp.float32))


def _h0(pos, W_in):
    return pl.pallas_call(
        _h0_body,
        grid=(N // _GB,),
        in_specs=[pl.BlockSpec((_GB, 3), lambda i: (i, 0)),
                  pl.BlockSpec((3, PS_OUT), lambda i: (0, 0))],
        out_specs=pl.BlockSpec((_GB, PS_OUT), lambda i: (i, 0)),
        out_shape=jax.ShapeDtypeStruct((N, PS_OUT), jnp.float32),
    )(pos, W_in)


def _gnn_layer_body(last, h_ref, agg_ref, d_ref, w_ref, wout_ref, o_ref,
                    p_out_ref):
    deg = jnp.maximum(d_ref[...], 1.0)
    agg = agg_ref[...] / deg
    hn = jnp.tanh(
        jax.lax.dot_general(h_ref[...] + agg, w_ref[...],
                            (((1,), (0,)), ((), ())),
                            preferred_element_type=jnp.float32))
    o_ref[...] = hn
    if last:
        p_out_ref[...] = jax.nn.sigmoid(
            jax.lax.dot_general(hn, wout_ref[...], (((1,), (0,)), ((), ())),
                                preferred_element_type=jnp.float32))


def _gnn_layer(h, agg_rows, deg_col, W, w_out, last):
    outs = [jax.ShapeDtypeStruct((N, PS_OUT), jnp.float32),
            jax.ShapeDtypeStruct((N, 1), jnp.float32)]
    return pl.pallas_call(
        functools.partial(_gnn_layer_body, last),
        grid=(N // _GB,),
        in_specs=[
            pl.BlockSpec((_GB, PS_OUT), lambda i: (i, 0)),
            pl.BlockSpec((_GB, PS_OUT), lambda i: (i, 0)),
            pl.BlockSpec((_GB, 1), lambda i: (i, 0)),
            pl.BlockSpec((PS_OUT, PS_OUT), lambda i: (0, 0)),
            pl.BlockSpec((PS_OUT, 1), lambda i: (0, 0)),
        ],
        out_specs=[pl.BlockSpec((_GB, PS_OUT), lambda i: (i, 0)),
                   pl.BlockSpec((_GB, 1), lambda i: (i, 0))],
        out_shape=outs,
    )(h, agg_rows, deg_col, W, w_out)


def _rank_body(p_ref, pt_ref, newid_ref):
    pi = p_ref[...]
    pj = pt_ref[...]
    i0 = pl.program_id(0) * _GB
    ii = i0 + jax.lax.broadcasted_iota(jnp.int32, (_GB, N), 0)
    jj = jax.lax.broadcasted_iota(jnp.int32, (_GB, N), 1)
    gt = (pj > pi) | ((pj == pi) & (jj < ii))
    rank = jnp.sum(gt.astype(jnp.int32), axis=1, keepdims=True)
    newid_ref[...] = jnp.where(rank < NS, rank, -1)


def _rank_newid(p_col, p_row):
    return pl.pallas_call(
        _rank_body,
        grid=(N // _GB,),
        in_specs=[pl.BlockSpec((_GB, 1), lambda i: (i, 0)),
                  pl.BlockSpec((1, N), lambda i: (0, 0))],
        out_specs=pl.BlockSpec((_GB, 1), lambda i: (i, 0)),
        out_shape=jax.ShapeDtypeStruct((N, 1), jnp.int32),
    )(p_col, p_row)


_RB = 200


def _topk_body(adj_ref, knn_ref, vals_ref):
    v = adj_ref[...]
    iota = jax.lax.broadcasted_iota(jnp.int32, (_RB, NS), 1)
    knn_cols, vals_cols = [], []
    for _ in range(K):
        m = jnp.max(v, axis=1, keepdims=True)
        idx = jnp.min(jnp.where(v == m, iota, jnp.int32(NS)), axis=1,
                      keepdims=True)
        knn_cols.append(idx)
        vals_cols.append(m)
        v = jnp.where(iota == idx, -jnp.inf, v)
    knn_ref[...] = jnp.concatenate(knn_cols, 1)
    vals_ref[...] = jnp.concatenate(vals_cols, 1)


def _row_topk(adj):
    return pl.pallas_call(
        _topk_body,
        grid=(NS // _RB,),
        in_specs=[pl.BlockSpec((_RB, NS), lambda i: (i, 0))],
        out_specs=[pl.BlockSpec((_RB, K), lambda i: (i, 0)),
                   pl.BlockSpec((_RB, K), lambda i: (i, 0))],
        out_shape=[jax.ShapeDtypeStruct((NS, K), jnp.int32),
                   jax.ShapeDtypeStruct((NS, K), jnp.float32)],
    )(adj)


_QK = [F // 2 - 1, F // 2]


def _quantile_body(fp_ref, mask_ref):
    v = fp_ref[...]
    u = jax.lax.bitcast_convert_type(v, jnp.uint32)

    def kth(k):
        def step(i, p):
            b = 31 - i
            mid = p | (jnp.uint32(1) << b.astype(jnp.uint32))
            c = jnp.sum((u < mid).astype(jnp.int32))
            return jnp.where(c > k, p, mid)
        return jax.lax.fori_loop(0, 32, step, jnp.uint32(0))

    lo = jax.lax.bitcast_convert_type(kth(_QK[0]), jnp.float32)
    hi = jax.lax.bitcast_convert_type(kth(_QK[1]), jnp.float32)
    thr = 0.5 * lo + 0.5 * hi
    mask_ref[...] = (v > thr).astype(jnp.int32)


def _quantile_mask(face_probs_padded):
    fp2 = face_probs_padded.reshape(FP // 128, 128)
    out = pl.pallas_call(
        _quantile_body,
        out_shape=jax.ShapeDtypeStruct((FP // 128, 128), jnp.int32),
    )(fp2)
    return out.reshape(FP)[:F]


def _face_mlp_body(x_ref, w_in_ref, b_in_ref, wh0_ref, bh0_ref, wh1_ref,
                   bh1_ref, w_out_ref, b_out_ref, o_ref):
    x = x_ref[...]
    fmask = x[10:11, :]
    dn = (((0,), (0,)), ((), ()))
    h = jax.lax.dot_general(w_in_ref[...], x, dn,
                            preferred_element_type=jnp.float32)
    h = jnp.maximum(h + b_in_ref[...].reshape(FC_HID, 1), 0.0)
    h = jax.lax.dot_general(wh0_ref[...], h, dn,
                            preferred_element_type=jnp.float32)
    h = jnp.maximum(h + bh0_ref[...].reshape(FC_HID, 1), 0.0)
    h = jax.lax.dot_general(wh1_ref[...], h, dn,
                            preferred_element_type=jnp.float32)
    h = jnp.maximum(h + bh1_ref[...].reshape(FC_HID, 1), 0.0)
    logit = jax.lax.dot_general(w_out_ref[...], h, dn,
                                preferred_element_type=jnp.float32)
    o_ref[...] = jax.nn.sigmoid(logit + b_out_ref[0, 0]) * fmask


def _face_mlp(x0t, Wf_in, bf_in, Wf_h, bf_h, Wf_out, bf_out):
    w_in_p = jnp.zeros((16, FC_HID), jnp.float32).at[:10].set(Wf_in)
    grid = (FP // _FBLK,)
    out = pl.pallas_call(
        _face_mlp_body,
        grid=grid,
        in_specs=[
            pl.BlockSpec((16, _FBLK), lambda i: (0, i)),
            pl.BlockSpec((16, FC_HID), lambda i: (0, 0)),
            pl.BlockSpec((FC_HID,), lambda i: (0,)),
            pl.BlockSpec((FC_HID, FC_HID), lambda i: (0, 0)),
            pl.BlockSpec((FC_HID,), lambda i: (0,)),
            pl.BlockSpec((FC_HID, FC_HID), lambda i: (0, 0)),
            pl.BlockSpec((FC_HID,), lambda i: (0,)),
            pl.BlockSpec((FC_HID, 1), lambda i: (0, 0)),
            pl.BlockSpec((1, 1), lambda i: (0, 0)),
        ],
        out_specs=pl.BlockSpec((1, _FBLK), lambda i: (0, i)),
        out_shape=jax.ShapeDtypeStruct((1, FP), jnp.float32),
    )(x0t, w_in_p, bf_in, Wf_h[0], bf_h[0], Wf_h[1], bf_h[1], Wf_out,
      bf_out.reshape(1, 1))
    return out[0]


def kernel(pos, edge_index, W_in, Ws, w_out, We1, be1, We2, Wf_in, bf_in,
           Wf_h, bf_h, Wf_out, bf_out):
    src = edge_index[0]
    dst = edge_index[1]
    h = _h0(pos, W_in)
    dp = _sc_deg(dst)
    deg_col = (dp[0, 0, :N] + dp[1, 0, :N]).reshape(N, 1)
    for i in range(PS_LAYERS):
        rows = _sc_gather_rows(h, src)
        agg_rows = jax.ops.segment_sum(rows, dst, num_segments=N)
        h, p_col = _gnn_layer(h, agg_rows, deg_col, Ws[i], w_out,
                              last=(i == PS_LAYERS - 1))
    sampled_probs = p_col[:, 0]
    new_id2 = _rank_newid(p_col, p_col.reshape(1, N))
    nid = new_id2[:, 0]
    valid_n = nid >= 0
    sampled_indices = jnp.zeros((NS,), jnp.int32).at[
        jnp.where(valid_n, nid, NS)].set(jnp.arange(N, dtype=jnp.int32),
                                         mode="drop")
    sampled_pos = pos[sampled_indices]
    s2 = nid[src]
    d2 = nid[dst]
    emask = ((s2 >= 0) & (d2 >= 0)).astype(jnp.float32)
    s2c = jnp.where(s2 >= 0, s2, 0)
    d2c = jnp.where(d2 >= 0, d2, 0)
    ef = jnp.concatenate([sampled_pos[s2c], sampled_pos[d2c]], axis=1)
    eprob = jax.nn.sigmoid(jax.nn.relu(ef @ We1 + be1) @ We2)[:, 0] * emask
    adj = jnp.zeros((NS, NS), jnp.float32).at[s2c, d2c].max(eprob)
    knn, vals = _row_topk(adj)
    jj = jnp.asarray(_JJ, jnp.int32)
    ll = jnp.asarray(_LL, jnp.int32)
    n1 = knn[:, jj]
    n2 = knn[:, ll]
    a12 = adj[n1, n2]
    tmask = (a12 > 0).astype(jnp.float32)
    tri_prob = (vals[:, jj] * vals[:, ll] * a12 / 3.0) * tmask
    ii = jnp.broadcast_to(jnp.arange(NS, dtype=jnp.int32)[:, None], n1.shape)
    faces = jnp.stack([ii, n1, n2], axis=-1).reshape(-1, 3)
    cprobs = tri_prob.reshape(-1)
    fmask = tmask.reshape(-1)
    fp = sampled_pos[faces].reshape(-1, 9)
    x0t = jnp.zeros((16, FP), jnp.float32)
    x0t = x0t.at[:9, :F].set(fp.T)
    x0t = x0t.at[9, :F].set(cprobs)
    x0t = x0t.at[10, :F].set(fmask)
    face_probs_p = _face_mlp(x0t, Wf_in, bf_in, Wf_h, bf_h, Wf_out, bf_out)
    face_probs = face_probs_p[:F]
    fp_pad = jnp.where(
        jnp.arange(FP, dtype=jnp.int32) < F, face_probs_p, jnp.inf)
    simplified_mask = _quantile_mask(fp_pad)
    return (sampled_probs, face_probs, simplified_mask)


# speedup vs baseline: 2.1608x; 1.7060x over previous
import functools

import jax
import jax.numpy as jnp
import numpy as np
from jax import lax
from jax.experimental import pallas as pl
from jax.experimental.pallas import tpu as pltpu
from jax.experimental.pallas import tpu_sc as plsc

N = 10000
E = 160000
RATIO = 0.5
K = 8
NS = int((1 - RATIO) * N)
PS_LAYERS = 3
PS_OUT = 128
EP_HID = 128
FC_HID = 256
FC_LAYERS = 2
_JJ, _LL = np.triu_indices(K, 1)
P = _JJ.shape[0]
F = NS * P

_FBLK = 1024
FP = ((F + _FBLK - 1) // _FBLK) * _FBLK


_NC, _NSUB, _NLANE = 2, 16, 16
_NW = _NC * _NSUB
_ECH = 640
_NCHUNK = E // _ECH
_NP = 10240
_RPT = _NP // _NSUB


_HALF = 5120
_HALFP = _HALF + 8
_RPT = _HALF // _NSUB
_CPT = _NCHUNK // _NSUB


def _gather_rows_body(h_hbm, src_hbm, out_hbm, idx_v, rows_v, sem):
    c = lax.axis_index("c")
    s = lax.axis_index("s")
    wid = s * _NC + c

    nch = jnp.where(wid < _NCHUNK % _NW, _NCHUNK // _NW + 1, _NCHUNK // _NW)

    def chunk_body(i, carry):
        base = (wid + i * _NW) * _ECH
        pltpu.sync_copy(src_hbm.at[pl.ds(base, _ECH)], idx_v)
        pltpu.async_copy(h_hbm.at[idx_v], rows_v, sem).wait()
        pltpu.sync_copy(rows_v, out_hbm.at[pl.ds(base, _ECH)])
        return carry

    lax.fori_loop(0, nch, chunk_body, 0)


def _sc_gather_rows(h, src):
    mesh = plsc.VectorSubcoreMesh(core_axis_name="c", subcore_axis_name="s")
    fn = pl.kernel(
        _gather_rows_body,
        out_type=jax.ShapeDtypeStruct((E, PS_OUT), jnp.float32),
        mesh=mesh,
        scratch_types=[
            pltpu.VMEM((_ECH,), jnp.int32),
            pltpu.VMEM((_ECH, PS_OUT), jnp.float32),
            pltpu.SemaphoreType.DMA,
        ],
    )
    return fn(h, src)


def _deg_body(dst_hbm, zdeg_hbm, ones_hbm, deg_hbm, dst_v, ones_v, degz_v,
              degacc_sh, sem):
    c = lax.axis_index("c")
    s = lax.axis_index("s")
    wid = s * _NC + c
    @pl.when(s == 0)
    def _():
        pltpu.sync_copy(zdeg_hbm, degz_v)
        pltpu.sync_copy(degz_v, degacc_sh)
    pltpu.sync_copy(ones_hbm, ones_v)
    plsc.subcore_barrier()

    nch = jnp.where(wid < _NCHUNK % _NW, _NCHUNK // _NW + 1, _NCHUNK // _NW)

    def chunk_body(i, carry):
        base = (wid + i * _NW) * _ECH
        pltpu.sync_copy(dst_hbm.at[pl.ds(base, _ECH)], dst_v)
        pltpu.sync_copy(ones_v, degacc_sh.at[dst_v], add=True)
        return carry

    lax.fori_loop(0, nch, chunk_body, 0)
    plsc.subcore_barrier()
    @pl.when(s == 0)
    def _():
        pltpu.sync_copy(degacc_sh, degz_v)
        pltpu.sync_copy(degz_v, deg_hbm.at[c, 0])


def _sc_deg(dst):
    mesh = plsc.VectorSubcoreMesh(core_axis_name="c", subcore_axis_name="s")
    zdeg = jnp.zeros((N + 16,), jnp.float32)
    ones = jnp.ones((_ECH,), jnp.float32)
    fn = pl.kernel(
        _deg_body,
        out_type=jax.ShapeDtypeStruct((2, 1, N + 16), jnp.float32),
        mesh=mesh,
        scratch_types=[
            pltpu.VMEM((_ECH,), jnp.int32),
            pltpu.VMEM((_ECH,), jnp.float32),
            pltpu.VMEM((N + 16,), jnp.float32),
            pltpu.VMEM_SHARED((N + 16,), jnp.float32),
            pltpu.SemaphoreType.DMA,
        ],
    )
    return fn(dst, zdeg, ones)


def _edge_feat_body(nid_hbm, posf_hbm, src_hbm, dst_hbm, eft_hbm, cell_hbm,
                    nid_v, pos_v, src_v, dst_v, eft_v, cell_v, sem):
    c = lax.axis_index("c")
    s = lax.axis_index("s")
    wid = s * _NC + c
    pltpu.sync_copy(nid_hbm, nid_v)
    pltpu.sync_copy(posf_hbm, pos_v)

    nch = jnp.where(wid < _NCHUNK % _NW, _NCHUNK // _NW + 1, _NCHUNK // _NW)

    def chunk_body(i, carry):
        base = (wid + i * _NW) * _ECH
        pltpu.sync_copy(src_hbm.at[pl.ds(base, _ECH)], src_v)
        pltpu.sync_copy(dst_hbm.at[pl.ds(base, _ECH)], dst_v)
        for g in range(_ECH // _NLANE):
            sl = pl.ds(g * _NLANE, _NLANE)
            s16 = src_v[sl]
            d16 = dst_v[sl]
            s2 = plsc.load_gather(nid_v, [s16])
            d2 = plsc.load_gather(nid_v, [d16])
            valid = (s2 >= 0) & (d2 >= 0)
            s2c = jnp.where(s2 >= 0, s2, 0)
            d2c = jnp.where(d2 >= 0, d2, 0)
            cell_v[sl] = jnp.where(valid, s2c * NS + d2c, 0)
            o = g * _NLANE
            for k in range(3):
                ps = plsc.load_gather(pos_v, [s16 * 3 + k])
                pdd = plsc.load_gather(pos_v, [d16 * 3 + k])
                eft_v[pl.ds(k * _ECH + o, _NLANE)] = ps
                eft_v[pl.ds((3 + k) * _ECH + o, _NLANE)] = pdd
            fmask = valid.astype(jnp.float32)
            eft_v[pl.ds(6 * _ECH + o, _NLANE)] = fmask
            eft_v[pl.ds(7 * _ECH + o, _NLANE)] = fmask * 0.0
        for k in range(8):
            pltpu.sync_copy(eft_v.at[pl.ds(k * _ECH, _ECH)],
                            eft_hbm.at[k, pl.ds(base, _ECH)])
        pltpu.sync_copy(cell_v, cell_hbm.at[pl.ds(base, _ECH)])
        return carry

    lax.fori_loop(0, nch, chunk_body, 0)


def _sc_edge_feat(nid_pad, pos_flat, src, dst):
    mesh = plsc.VectorSubcoreMesh(core_axis_name="c", subcore_axis_name="s")
    fn = pl.kernel(
        _edge_feat_body,
        out_type=(jax.ShapeDtypeStruct((8, E), jnp.float32),
                  jax.ShapeDtypeStruct((E,), jnp.int32)),
        mesh=mesh,
        compiler_params=pltpu.CompilerParams(needs_layout_passes=False),
        scratch_types=[
            pltpu.VMEM((10240,), jnp.int32),
            pltpu.VMEM((N * 3,), jnp.float32),
            pltpu.VMEM((_ECH,), jnp.int32),
            pltpu.VMEM((_ECH,), jnp.int32),
            pltpu.VMEM((8 * _ECH,), jnp.float32),
            pltpu.VMEM((_ECH,), jnp.int32),
            pltpu.SemaphoreType.DMA,
        ],
    )
    return fn(nid_pad, pos_flat, src, dst)


_EB = 1280


def _edge_mlp_body(x_ref, w1_ref, b1_ref, w2_ref, o_ref):
    x = x_ref[...]
    emask = x[6:7, :]
    dn = (((0,), (0,)), ((), ()))
    h = jnp.maximum(
        jax.lax.dot_general(w1_ref[...], x, dn,
                            preferred_element_type=jnp.float32)
        + b1_ref[...].reshape(EP_HID, 1), 0.0)
    logit = jax.lax.dot_general(w2_ref[...], h, dn,
                                preferred_element_type=jnp.float32)
    o_ref[...] = jax.nn.sigmoid(logit) * emask


def _edge_mlp(eft, We1p, be1, We2):
    out = pl.pallas_call(
        _edge_mlp_body,
        grid=(E // _EB,),
        in_specs=[
            pl.BlockSpec((8, _EB), lambda i: (0, i)),
            pl.BlockSpec((8, EP_HID), lambda i: (0, 0)),
            pl.BlockSpec((EP_HID,), lambda i: (0,)),
            pl.BlockSpec((EP_HID, 1), lambda i: (0, 0)),
        ],
        out_specs=pl.BlockSpec((1, _EB), lambda i: (0, i)),
        out_shape=jax.ShapeDtypeStruct((1, E), jnp.float32),
    )(eft, We1p, be1, We2)
    return out[0]


_FCH = 512
_NFCH = FP // _FCH
_KROWS = 40


def _face_feat_body(sp_hbm, knn_hbm, vals_hbm, adj_hbm, jj_hbm, ll_hbm,
                    x0t_hbm, sp_v, knn_v, vals_v, jj_v, ll_v, out_v, aidx_v,
                    aval_v, sem, sem2):
    c = lax.axis_index("c")
    s = lax.axis_index("s")
    wid = s * _NC + c
    pltpu.sync_copy(sp_hbm, sp_v)
    pltpu.sync_copy(jj_hbm, jj_v)
    pltpu.sync_copy(ll_hbm, ll_v)

    nch = jnp.where(wid < _NFCH % _NW, _NFCH // _NW + 1, _NFCH // _NW)
    inv28 = jnp.float32(1.0 / 28.0)

    def chunk_body(i, carry):
        ch = wid + i * _NW
        fbase = ch * _FCH
        rb8 = (fbase // 28) // 8 * 8
        pltpu.sync_copy(knn_hbm.at[pl.ds(rb8, _KROWS)], knn_v)
        pltpu.sync_copy(vals_hbm.at[pl.ds(rb8, _KROWS)], vals_v)
        for g in range(_FCH // _NLANE):
            sl = pl.ds(g * _NLANE, _NLANE)
            f = fbase + g * _NLANE + lax.iota(jnp.int32, _NLANE)
            r = ((f.astype(jnp.float32) + 0.5) * inv28).astype(jnp.int32)
            r = jnp.minimum(r, NS - 1)
            p = f - r * 28
            jjp = plsc.load_gather(jj_v, [p])
            llp = plsc.load_gather(ll_v, [p])
            rloc = r - rb8
            n1 = plsc.load_gather(knn_v, [rloc, jjp])
            n2 = plsc.load_gather(knn_v, [rloc, llp])
            aidx_v[sl] = n1 * NS + n2
        cps = []
        for b in range(_FCH // 128):
            cps.append(pltpu.async_copy(
                adj_hbm.at[aidx_v.at[pl.ds(b * 128, 128)]],
                aval_v.at[pl.ds(b * 128, 128)], sem2))
        for cp in cps:
            cp.wait()
        for g in range(_FCH // _NLANE):
            sl = pl.ds(g * _NLANE, _NLANE)
            f = fbase + g * _NLANE + lax.iota(jnp.int32, _NLANE)
            r = ((f.astype(jnp.float32) + 0.5) * inv28).astype(jnp.int32)
            r = jnp.minimum(r, NS - 1)
            p = f - r * 28
            jjp = plsc.load_gather(jj_v, [p])
            llp = plsc.load_gather(ll_v, [p])
            rloc = r - rb8
            n1 = plsc.load_gather(knn_v, [rloc, jjp])
            n2 = plsc.load_gather(knn_v, [rloc, llp])
            v1 = plsc.load_gather(vals_v, [rloc, jjp])
            v2 = plsc.load_gather(vals_v, [rloc, llp])
            a12 = aval_v[sl]
            tmask = (a12 > 0).astype(jnp.float32)
            tri = (v1 * v2 * a12 / 3.0) * tmask
            fok = (f < F).astype(jnp.float32)
            for k in range(3):
                out_v[k, sl] = plsc.load_gather(sp_v, [r * 3 + k])
                out_v[3 + k, sl] = plsc.load_gather(sp_v, [n1 * 3 + k])
                out_v[6 + k, sl] = plsc.load_gather(sp_v, [n2 * 3 + k])
            out_v[9, sl] = tri
            out_v[10, sl] = tmask * fok
            z = jnp.zeros((_NLANE,), jnp.float32)
            for k in range(11, 16):
                out_v[k, sl] = z
        pltpu.sync_copy(out_v, x0t_hbm.at[:, pl.ds(fbase, _FCH)])
        return carry

    lax.fori_loop(0, nch, chunk_body, 0)


def _sc_face_feat(sp_flat, knn_pad, vals_pad, adj_flat, jj, ll):
    mesh = plsc.VectorSubcoreMesh(core_axis_name="c", subcore_axis_name="s")
    fn = pl.kernel(
        _face_feat_body,
        out_type=jax.ShapeDtypeStruct((16, FP), jnp.float32),
        mesh=mesh,
        compiler_params=pltpu.CompilerParams(needs_layout_passes=False),
        scratch_types=[
            pltpu.VMEM((NS * 3,), jnp.float32),
            pltpu.VMEM((_KROWS, K), jnp.int32),
            pltpu.VMEM((_KROWS, K), jnp.float32),
            pltpu.VMEM((32,), jnp.int32),
            pltpu.VMEM((32,), jnp.int32),
            pltpu.VMEM((16, _FCH), jnp.float32),
            pltpu.VMEM((_FCH,), jnp.int32),
            pltpu.VMEM((_FCH,), jnp.float32),
            pltpu.SemaphoreType.DMA,
            pltpu.SemaphoreType.DMA,
        ],
    )
    return fn(sp_flat, knn_pad, vals_pad, adj_flat, jj, ll)


_GB = 1000


def _h0_body(pos_ref, w_ref, o_ref):
    o_ref[...] = jnp.tanh(
        jax.lax.dot_general(pos_ref[...], w_ref[...], (((1,), (0,)), ((), ())),
                            preferred_element_type=jnp.float32))


def _h0(pos, W_in):
    return pl.pallas_call(
        _h0_body,
        grid=(N // _GB,),
        in_specs=[pl.BlockSpec((_GB, 3), lambda i: (i, 0)),
                  pl.BlockSpec((3, PS_OUT), lambda i: (0, 0))],
        out_specs=pl.BlockSpec((_GB, PS_OUT), lambda i: (i, 0)),
        out_shape=jax.ShapeDtypeStruct((N, PS_OUT), jnp.float32),
    )(pos, W_in)


def _gnn_layer_body(last, h_ref, agg_ref, d_ref, w_ref, wout_ref, o_ref,
                    p_out_ref):
    deg = jnp.maximum(d_ref[...], 1.0)
    agg = agg_ref[...] / deg
    hn = jnp.tanh(
        jax.lax.dot_general(h_ref[...] + agg, w_ref[...],
                            (((1,), (0,)), ((), ())),
                            preferred_element_type=jnp.float32))
    o_ref[...] = hn
    if last:
        p_out_ref[...] = jax.nn.sigmoid(
            jax.lax.dot_general(hn, wout_ref[...], (((1,), (0,)), ((), ())),
                                preferred_element_type=jnp.float32))


def _gnn_layer(h, agg_rows, deg_col, W, w_out, last):
    outs = [jax.ShapeDtypeStruct((N, PS_OUT), jnp.float32),
            jax.ShapeDtypeStruct((N, 1), jnp.float32)]
    return pl.pallas_call(
        functools.partial(_gnn_layer_body, last),
        grid=(N // _GB,),
        in_specs=[
            pl.BlockSpec((_GB, PS_OUT), lambda i: (i, 0)),
            pl.BlockSpec((_GB, PS_OUT), lambda i: (i, 0)),
            pl.BlockSpec((_GB, 1), lambda i: (i, 0)),
            pl.BlockSpec((PS_OUT, PS_OUT), lambda i: (0, 0)),
            pl.BlockSpec((PS_OUT, 1), lambda i: (0, 0)),
        ],
        out_specs=[pl.BlockSpec((_GB, PS_OUT), lambda i: (i, 0)),
                   pl.BlockSpec((_GB, 1), lambda i: (i, 0))],
        out_shape=outs,
    )(h, agg_rows, deg_col, W, w_out)


def _rank_body(p_ref, pt_ref, newid_ref):
    pi = p_ref[...]
    pj = pt_ref[...]
    i0 = pl.program_id(0) * _GB
    ii = i0 + jax.lax.broadcasted_iota(jnp.int32, (_GB, N), 0)
    jj = jax.lax.broadcasted_iota(jnp.int32, (_GB, N), 1)
    gt = (pj > pi) | ((pj == pi) & (jj < ii))
    rank = jnp.sum(gt.astype(jnp.int32), axis=1, keepdims=True)
    newid_ref[...] = jnp.where(rank < NS, rank, -1)


def _rank_newid(p_col, p_row):
    return pl.pallas_call(
        _rank_body,
        grid=(N // _GB,),
        in_specs=[pl.BlockSpec((_GB, 1), lambda i: (i, 0)),
                  pl.BlockSpec((1, N), lambda i: (0, 0))],
        out_specs=pl.BlockSpec((_GB, 1), lambda i: (i, 0)),
        out_shape=jax.ShapeDtypeStruct((N, 1), jnp.int32),
    )(p_col, p_row)


_RB = 200


def _topk_body(adj_ref, knn_ref, vals_ref):
    v = adj_ref[...]
    iota = jax.lax.broadcasted_iota(jnp.int32, (_RB, NS), 1)
    knn_cols, vals_cols = [], []
    for _ in range(K):
        m = jnp.max(v, axis=1, keepdims=True)
        idx = jnp.min(jnp.where(v == m, iota, jnp.int32(NS)), axis=1,
                      keepdims=True)
        knn_cols.append(idx)
        vals_cols.append(m)
        v = jnp.where(iota == idx, -jnp.inf, v)
    knn_ref[...] = jnp.concatenate(knn_cols, 1)
    vals_ref[...] = jnp.concatenate(vals_cols, 1)


def _row_topk(adj):
    return pl.pallas_call(
        _topk_body,
        grid=(NS // _RB,),
        in_specs=[pl.BlockSpec((_RB, NS), lambda i: (i, 0))],
        out_specs=[pl.BlockSpec((_RB, K), lambda i: (i, 0)),
                   pl.BlockSpec((_RB, K), lambda i: (i, 0))],
        out_shape=[jax.ShapeDtypeStruct((NS, K), jnp.int32),
                   jax.ShapeDtypeStruct((NS, K), jnp.float32)],
    )(adj)


_QK = [F // 2 - 1, F // 2]


def _quantile_body(fp_ref, mask_ref):
    v = fp_ref[...]
    u = jax.lax.bitcast_convert_type(v, jnp.uint32)

    def kth(k):
        def step(i, p):
            b = 31 - i
            mid = p | (jnp.uint32(1) << b.astype(jnp.uint32))
            c = jnp.sum((u < mid).astype(jnp.int32))
            return jnp.where(c > k, p, mid)
        return jax.lax.fori_loop(0, 32, step, jnp.uint32(0))

    lo = jax.lax.bitcast_convert_type(kth(_QK[0]), jnp.float32)
    hi = jax.lax.bitcast_convert_type(kth(_QK[1]), jnp.float32)
    thr = 0.5 * lo + 0.5 * hi
    mask_ref[...] = (v > thr).astype(jnp.int32)


def _quantile_mask(face_probs_padded):
    fp2 = face_probs_padded.reshape(FP // 128, 128)
    out = pl.pallas_call(
        _quantile_body,
        out_shape=jax.ShapeDtypeStruct((FP // 128, 128), jnp.int32),
    )(fp2)
    return out.reshape(FP)[:F]


def _face_mlp_body(x_ref, w_in_ref, b_in_ref, wh0_ref, bh0_ref, wh1_ref,
                   bh1_ref, w_out_ref, b_out_ref, o_ref):
    x = x_ref[...]
    fmask = x[10:11, :]
    dn = (((0,), (0,)), ((), ()))
    h = jax.lax.dot_general(w_in_ref[...], x, dn,
                            preferred_element_type=jnp.float32)
    h = jnp.maximum(h + b_in_ref[...].reshape(FC_HID, 1), 0.0)
    h = jax.lax.dot_general(wh0_ref[...], h, dn,
                            preferred_element_type=jnp.float32)
    h = jnp.maximum(h + bh0_ref[...].reshape(FC_HID, 1), 0.0)
    h = jax.lax.dot_general(wh1_ref[...], h, dn,
                            preferred_element_type=jnp.float32)
    h = jnp.maximum(h + bh1_ref[...].reshape(FC_HID, 1), 0.0)
    logit = jax.lax.dot_general(w_out_ref[...], h, dn,
                                preferred_element_type=jnp.float32)
    o_ref[...] = jax.nn.sigmoid(logit + b_out_ref[0, 0]) * fmask


def _face_mlp(x0t, Wf_in, bf_in, Wf_h, bf_h, Wf_out, bf_out):
    w_in_p = jnp.zeros((16, FC_HID), jnp.float32).at[:10].set(Wf_in)
    grid = (FP // _FBLK,)
    out = pl.pallas_call(
        _face_mlp_body,
        grid=grid,
        in_specs=[
            pl.BlockSpec((16, _FBLK), lambda i: (0, i)),
            pl.BlockSpec((16, FC_HID), lambda i: (0, 0)),
            pl.BlockSpec((FC_HID,), lambda i: (0,)),
            pl.BlockSpec((FC_HID, FC_HID), lambda i: (0, 0)),
            pl.BlockSpec((FC_HID,), lambda i: (0,)),
            pl.BlockSpec((FC_HID, FC_HID), lambda i: (0, 0)),
            pl.BlockSpec((FC_HID,), lambda i: (0,)),
            pl.BlockSpec((FC_HID, 1), lambda i: (0, 0)),
            pl.BlockSpec((1, 1), lambda i: (0, 0)),
        ],
        out_specs=pl.BlockSpec((1, _FBLK), lambda i: (0, i)),
        out_shape=jax.ShapeDtypeStruct((1, FP), jnp.float32),
    )(x0t, w_in_p, bf_in, Wf_h[0], bf_h[0], Wf_h[1], bf_h[1], Wf_out,
      bf_out.reshape(1, 1))
    return out[0]


def kernel(pos, edge_index, W_in, Ws, w_out, We1, be1, We2, Wf_in, bf_in,
           Wf_h, bf_h, Wf_out, bf_out):
    src = edge_index[0]
    dst = edge_index[1]
    h = _h0(pos, W_in)
    dp = _sc_deg(dst)
    deg_col = (dp[0, 0, :N] + dp[1, 0, :N]).reshape(N, 1)
    for i in range(PS_LAYERS):
        rows = _sc_gather_rows(h, src)
        agg_rows = jax.ops.segment_sum(rows, dst, num_segments=N)
        h, p_col = _gnn_layer(h, agg_rows, deg_col, Ws[i], w_out,
                              last=(i == PS_LAYERS - 1))
    sampled_probs = p_col[:, 0]
    new_id2 = _rank_newid(p_col, p_col.reshape(1, N))
    nid = new_id2[:, 0]
    valid_n = nid >= 0
    sampled_indices = jnp.zeros((NS,), jnp.int32).at[
        jnp.where(valid_n, nid, NS)].set(jnp.arange(N, dtype=jnp.int32),
                                         mode="drop")
    sampled_pos = pos[sampled_indices]
    nid_pad = jnp.concatenate([nid, jnp.full((10240 - N,), -1, jnp.int32)])
    eft, cell = _sc_edge_feat(nid_pad, pos.reshape(-1), src, dst)
    We1p = jnp.zeros((8, EP_HID), jnp.float32).at[:6].set(We1)
    eprob = _edge_mlp(eft, We1p, be1, We2)
    adj = jnp.zeros((NS, NS), jnp.float32).at[cell // NS, cell % NS].max(
        eprob)
    knn, vals = _row_topk(adj)
    knn_pad = jnp.zeros((5040, K), jnp.int32).at[:NS].set(knn)
    vals_pad = jnp.zeros((5040, K), jnp.float32).at[:NS].set(vals)
    jjp = jnp.zeros((32,), jnp.int32).at[:P].set(jnp.asarray(_JJ, jnp.int32))
    llp = jnp.zeros((32,), jnp.int32).at[:P].set(jnp.asarray(_LL, jnp.int32))
    jj = jnp.asarray(_JJ, jnp.int32)
    ll = jnp.asarray(_LL, jnp.int32)
    n1 = knn[:, jj]
    n2 = knn[:, ll]
    a12 = adj[n1, n2]
    tmask = (a12 > 0).astype(jnp.float32)
    tri_prob = (vals[:, jj] * vals[:, ll] * a12 / 3.0) * tmask
    ii = jnp.broadcast_to(jnp.arange(NS, dtype=jnp.int32)[:, None], n1.shape)
    faces = jnp.stack([ii, n1, n2], axis=-1).reshape(-1, 3)
    fp = sampled_pos[faces].reshape(-1, 9)
    x0t = jnp.zeros((16, FP), jnp.float32)
    x0t = x0t.at[:9, :F].set(fp.T)
    x0t = x0t.at[9, :F].set(tri_prob.reshape(-1))
    x0t = x0t.at[10, :F].set(tmask.reshape(-1))
    face_probs_p = _face_mlp(x0t, Wf_in, bf_in, Wf_h, bf_h, Wf_out, bf_out)
    face_probs = face_probs_p[:F]
    fp_pad = jnp.where(
        jnp.arange(FP, dtype=jnp.int32) < F, face_probs_p, jnp.inf)
    simplified_mask = _quantile_mask(fp_pad)
    return (sampled_probs, face_probs, simplified_mask)
